# baseline (speedup 1.0000x reference)
import functools

import jax
import jax.numpy as jnp
from jax import lax
from jax.experimental import pallas as pl
from jax.experimental.pallas import tpu as pltpu
from jax.experimental.pallas import tpu_sc as plsc

LANES = 16
NUM_WORKERS = 32
IDX_CHUNK = 128
GROUP = 16


def _make_gather_kernel(W, B, V, D):
  b_per_w = B // NUM_WORKERS
  n_blocks = b_per_w // LANES
  copies = b_per_w // IDX_CHUNK
  mesh = plsc.VectorSubcoreMesh(core_axis_name="c", subcore_axis_name="s")

  @functools.partial(
      pl.kernel,
      out_type=(
          jax.ShapeDtypeStruct((W, D, B), jnp.float32),
          jax.ShapeDtypeStruct((W, B), jnp.float32),
          jax.ShapeDtypeStruct((B,), jnp.int32),
      ),
      mesh=mesh,
      compiler_params=pltpu.CompilerParams(
          needs_layout_passes=False, use_tc_tiling_on_sc=False,
          disable_bounds_checks=True),
      scratch_types=[
          pltpu.VMEM((W, b_per_w), jnp.int32),
          pltpu.VMEM((W, b_per_w), jnp.float32),
          pltpu.VMEM((b_per_w,), jnp.int32),
          pltpu.VMEM((b_per_w,), jnp.int32),
          pltpu.VMEM((W, b_per_w), jnp.int32),
          pltpu.VMEM((W, b_per_w), jnp.float32),
          pltpu.VMEM((W, b_per_w), jnp.float32),
          pltpu.VMEM((D, b_per_w), jnp.float32),
          pltpu.SemaphoreType.DMA,
      ],
  )
  def kb(tt_hbm, rew_hbm, idx_hbm, len_hbm,
         seq_hbm, mask_hbm, lens_hbm,
         idx_v, rew_v, len_v, lenc_v, gid_v, scale_v, mask_v,
         panel_v, sem):
    wid = lax.axis_index("s") * 2 + lax.axis_index("c")
    b0 = wid * b_per_w

    pltpu.sync_copy(idx_hbm.at[:, pl.ds(b0, b_per_w)], idx_v)
    pltpu.sync_copy(rew_hbm.at[:, pl.ds(b0, b_per_w)], rew_v)
    pltpu.sync_copy(len_hbm.at[pl.ds(b0, b_per_w)], len_v)

    def blk_body(blk, carry):
      bi = blk * LANES + jnp.arange(LANES, dtype=jnp.int32)
      L = len_v[pl.ds(blk * LANES, LANES)]
      Lc = jnp.clip(L, 1, W)
      lenc_v[pl.ds(blk * LANES, LANES)] = jnp.clip(L, 0, W)
      for j in range(W):
        tj = jnp.where(j < Lc, Lc - 1 - j, j)
        g = plsc.load_gather(idx_v, [tj, bi])
        g = jnp.where(g == -1, V - 1, g)
        g = jnp.clip(g, 0, V - 1)
        r = plsc.load_gather(rew_v, [tj, bi])
        live = j < L
        m = jnp.where(live, jnp.float32(1.0), jnp.float32(0.0))
        s = jnp.minimum(r, jnp.float32(1.0)) * m
        gid_v[j, pl.ds(blk * LANES, LANES)] = g
        scale_v[j, pl.ds(blk * LANES, LANES)] = s
        mask_v[j, pl.ds(blk * LANES, LANES)] = m
      return carry

    lax.fori_loop(0, n_blocks, blk_body, 0)

    pltpu.sync_copy(mask_v, mask_hbm.at[:, pl.ds(b0, b_per_w)])
    pltpu.sync_copy(lenc_v, lens_hbm.at[pl.ds(b0, b_per_w)])

    n_pairs = D * copies

    def j_body(j, carry):
      def issue_group(grp):
        for u in range(GROUP):
          pair = grp * GROUP + u
          d = pair // copies
          k = pair % copies
          pltpu.async_copy(
              tt_hbm.at[d].at[gid_v.at[j, pl.ds(k * IDX_CHUNK, IDX_CHUNK)]],
              panel_v.at[d, pl.ds(k * IDX_CHUNK, IDX_CHUNK)],
              sem)

      def drain_group():
        for _ in range(GROUP):
          pltpu.make_async_copy(
              tt_hbm.at[0].at[gid_v.at[j, pl.ds(0, IDX_CHUNK)]],
              panel_v.at[0, pl.ds(0, IDX_CHUNK)],
              sem).wait()

      issue_group(0)
      def grp_body(grp, gcarry):
        issue_group(grp + 1)
        drain_group()
        return gcarry
      lax.fori_loop(0, n_pairs // GROUP - 1, grp_body, 0)
      drain_group()

      def vg_body(vg, vcarry):
        sv = scale_v[j, pl.ds(vg * LANES, LANES)]
        for d in range(D):
          panel_v[d, pl.ds(vg * LANES, LANES)] = (
              panel_v[d, pl.ds(vg * LANES, LANES)] * sv)
        return vcarry

      lax.fori_loop(0, n_blocks, vg_body, 0)
      pltpu.sync_copy(panel_v, seq_hbm.at[j, :, pl.ds(b0, b_per_w)])
      return carry

    lax.fori_loop(0, W, j_body, 0)

  return kb


def kernel(item_table, rewards, item_indices, lengths):
  W, B = item_indices.shape
  V, D = item_table.shape
  kb = _make_gather_kernel(W, B, V, D)
  tt = jnp.transpose(item_table)
  seq_t, mask_t, len_states = kb(
      tt, rewards, item_indices.astype(jnp.int32),
      lengths.astype(jnp.int32))
  seq = jnp.transpose(seq_t, (2, 0, 1))
  mask_bw = jnp.transpose(mask_t)[:, :, None]
  return seq, mask_bw, len_states


# speedup vs baseline: 4.4393x; 4.4393x over previous
import functools

import jax
import jax.numpy as jnp
from jax import lax
from jax.experimental import pallas as pl
from jax.experimental.pallas import tpu as pltpu
from jax.experimental.pallas import tpu_sc as plsc

LANES = 16
NUM_WORKERS = 32
IDX_CHUNK = 128
GATHER_CHUNK = 512


def _make_sc_kernel(W, B, V, D):
  b_per_w = B // NUM_WORKERS
  rows_per_w = b_per_w * W
  n_blocks = b_per_w // LANES
  n_chunks = rows_per_w // GATHER_CHUNK
  copies_per_chunk = GATHER_CHUNK // IDX_CHUNK
  mesh = plsc.VectorSubcoreMesh(core_axis_name="c", subcore_axis_name="s")

  @functools.partial(
      pl.kernel,
      out_type=(
          jax.ShapeDtypeStruct((B * W, D), jnp.float32),
          jax.ShapeDtypeStruct((B * W,), jnp.float32),
          jax.ShapeDtypeStruct((B,), jnp.int32),
      ),
      mesh=mesh,
      compiler_params=pltpu.CompilerParams(
          needs_layout_passes=False, use_tc_tiling_on_sc=False,
          disable_bounds_checks=True),
      scratch_types=[
          pltpu.VMEM((W, b_per_w), jnp.int32),
          pltpu.VMEM((W, b_per_w), jnp.float32),
          pltpu.VMEM((b_per_w,), jnp.int32),
          pltpu.VMEM((b_per_w,), jnp.int32),
          pltpu.VMEM((rows_per_w,), jnp.int32),
          pltpu.VMEM((rows_per_w,), jnp.float32),
          pltpu.VMEM((rows_per_w,), jnp.float32),
          pltpu.VMEM((GATHER_CHUNK, D), jnp.float32),
          pltpu.SemaphoreType.DMA,
      ],
  )
  def sc_kernel(table_hbm, rew_hbm, idx_hbm, len_hbm,
                seq_hbm, mask_hbm, lens_hbm,
                idx_v, rew_v, len_v, lenc_v, gid_v, scale_v, mask_v,
                rows_v, sem):
    wid = lax.axis_index("s") * 2 + lax.axis_index("c")
    b0 = wid * b_per_w
    row0 = wid * rows_per_w

    pltpu.sync_copy(idx_hbm.at[:, pl.ds(b0, b_per_w)], idx_v)
    pltpu.sync_copy(rew_hbm.at[:, pl.ds(b0, b_per_w)], rew_v)
    pltpu.sync_copy(len_hbm.at[pl.ds(b0, b_per_w)], len_v)

    def blk_body(blk, carry):
      bi = blk * LANES + jnp.arange(LANES, dtype=jnp.int32)
      L = len_v[pl.ds(blk * LANES, LANES)]
      Lc = jnp.clip(L, 1, W)
      lenc_v[pl.ds(blk * LANES, LANES)] = jnp.clip(L, 0, W)
      for j in range(W):
        tj = jnp.where(j < Lc, Lc - 1 - j, j)
        g = plsc.load_gather(idx_v, [tj, bi])
        g = jnp.where(g == -1, V - 1, g)
        g = jnp.clip(g, 0, V - 1)
        r = plsc.load_gather(rew_v, [tj, bi])
        live = j < L
        m = jnp.where(live, jnp.float32(1.0), jnp.float32(0.0))
        s = jnp.minimum(r, jnp.float32(1.0)) * m
        pos = bi * W + j
        plsc.store_scatter(gid_v, [pos], g)
        plsc.store_scatter(scale_v, [pos], s)
        plsc.store_scatter(mask_v, [pos], m)
      return carry

    lax.fori_loop(0, n_blocks, blk_body, 0)

    pltpu.sync_copy(mask_v, mask_hbm.at[pl.ds(row0, rows_per_w)])
    pltpu.sync_copy(lenc_v, lens_hbm.at[pl.ds(b0, b_per_w)])

    def chunk_body(c, carry):
      r0 = c * GATHER_CHUNK
      cps = []
      for k in range(copies_per_chunk):
        cps.append(pltpu.async_copy(
            table_hbm.at[gid_v.at[pl.ds(r0 + k * IDX_CHUNK, IDX_CHUNK)]],
            rows_v.at[pl.ds(k * IDX_CHUNK, IDX_CHUNK)],
            sem))
      for cp in cps:
        cp.wait()

      def grp_body(g, rcarry):
        rbase = g * LANES
        sv = scale_v[pl.ds(r0 + rbase, LANES)]
        for i in range(LANES):
          s = sv[i]
          for h in range(D // LANES):
            seg = rows_v[rbase + i, pl.ds(h * LANES, LANES)]
            rows_v[rbase + i, pl.ds(h * LANES, LANES)] = seg * s
        return rcarry

      lax.fori_loop(0, GATHER_CHUNK // LANES, grp_body, 0)
      pltpu.sync_copy(rows_v, seq_hbm.at[pl.ds(row0 + r0, GATHER_CHUNK)])
      return carry

    lax.fori_loop(0, n_chunks, chunk_body, 0)

  return sc_kernel


def kernel(item_table, rewards, item_indices, lengths):
  W, B = item_indices.shape
  V, D = item_table.shape
  sc = _make_sc_kernel(W, B, V, D)
  seq_rows, mask_flat, len_states = sc(
      item_table, rewards, item_indices.astype(jnp.int32),
      lengths.astype(jnp.int32))
  seq = seq_rows.reshape(B, W, D)
  mask_bw = mask_flat.reshape(B, W, 1)
  return seq, mask_bw, len_states
